# Initial kernel scaffold; baseline (speedup 1.0000x reference)
import jax
import jax.numpy as jnp
from jax.experimental import pallas as pl


def kernel(x_Entity, x_Transaction, edge_index_e2t, edge_index_t2e, edge_weight_e2t, edge_weight_t2e, l1_e2t_Wrel, l1_e2t_brel, l1_e2t_Wroot, l1_t2e_Wrel, l1_t2e_brel, l1_t2e_Wroot, l2_e2t_Wrel, l2_e2t_brel, l2_e2t_Wroot, l2_t2e_Wrel, l2_t2e_brel, l2_t2e_Wroot, l3_e2t_Wrel, l3_e2t_brel, l3_e2t_Wroot, l3_t2e_Wrel, l3_t2e_brel, l3_t2e_Wroot, lin1_W, lin1_b, lin2_W, lin2_b):
    raise NotImplementedError("write your pallas kernel here")


# speedup vs baseline: 1.9572x; 1.9572x over previous
import functools

import jax
import jax.numpy as jnp
from jax import lax
from jax.experimental import pallas as pl
from jax.experimental.pallas import tpu as pltpu
from jax.experimental.pallas import tpu_sc as plsc

N = 10000
NP = 10240
E = 160000
LANES = 16
NSUB = 16
NCORE = 2
EPT = E // NSUB
EB = 80
NBATCH = EPT // EB
RPT = NP // NSUB
ZROWS = 128
CW = 128


_GATHER_DNUMS = lax.GatherDimensionNumbers(
    offset_dims=(), collapsed_slice_dims=(0,), start_index_map=(0,))


def _splat(vec, j):
    idx = jnp.full((LANES, 1), j, jnp.int32)
    return lax.gather(vec, idx, _GATHER_DNUMS, (1,),
                      mode=lax.GatherScatterMode.PROMISE_IN_BOUNDS)


def _scale_rows(rows, ewv):
    def group(g, carry):
        ewreg = ewv[pl.ds(g * LANES, LANES)]
        for j in range(LANES):
            sp = _splat(ewreg, j)
            i = g * LANES + j
            for k in range(CW // LANES):
                sl = pl.ds(k * LANES, LANES)
                rows[i, sl] = rows[i, sl] * sp
        return carry
    lax.fori_loop(0, EB // LANES, group, 0)


def _one_pass(xc, outc, src_r, dst_r, ew_r, acc, zbuf, rows, srcv, dstv,
              ewv, sid):
    row0 = sid * RPT
    for z in range(RPT // ZROWS):
        pltpu.sync_copy(zbuf, acc.at[pl.ds(row0 + z * ZROWS, ZROWS)])
    plsc.subcore_barrier()
    ebase = sid * EPT

    def batch(b, carry):
        base = ebase + b * EB
        pltpu.sync_copy(src_r.at[pl.ds(base, EB)], srcv)
        pltpu.sync_copy(dst_r.at[pl.ds(base, EB)], dstv)
        pltpu.sync_copy(ew_r.at[pl.ds(base, EB)], ewv)
        pltpu.sync_copy(xc.at[srcv], rows)
        _scale_rows(rows, ewv)
        pltpu.sync_copy(rows, acc.at[dstv], add=True)
        return carry

    lax.fori_loop(0, NBATCH, batch, 0)
    plsc.subcore_barrier()
    pltpu.sync_copy(acc.at[pl.ds(row0, RPT)], outc.at[pl.ds(row0, RPT)])
    plsc.subcore_barrier()


def _segment_sum_sc(xchunks, src, dst, ew):
    nc = len(xchunks)
    npc = nc // NCORE
    mesh = plsc.VectorSubcoreMesh(core_axis_name="c", subcore_axis_name="s")
    out_type = tuple(
        jax.ShapeDtypeStruct((NP, CW), jnp.float32) for _ in range(nc))
    scratch = [
        pltpu.VMEM_SHARED((NP, CW), jnp.float32),
        pltpu.VMEM((ZROWS, CW), jnp.float32),
        pltpu.VMEM((EB, CW), jnp.float32),
        pltpu.VMEM((EB,), jnp.int32),
        pltpu.VMEM((EB,), jnp.int32),
        pltpu.VMEM((EB,), jnp.float32),
    ]

    def body(*refs):
        xs = refs[:nc]
        src_r, dst_r, ew_r = refs[nc:nc + 3]
        outs = refs[nc + 3:2 * nc + 3]
        acc, zbuf, rows, srcv, dstv, ewv = refs[2 * nc + 3:]
        cid = lax.axis_index("c")
        sid = lax.axis_index("s")

        def zb(i, carry):
            for k in range(CW // LANES):
                zbuf[i, pl.ds(k * LANES, LANES)] = jnp.zeros(
                    (LANES,), jnp.float32)
            return carry
        lax.fori_loop(0, ZROWS, zb, 0)

        for j in range(npc):
            for core in range(NCORE):
                chunk = core * npc + j

                @pl.when(cid == core)
                def _(chunk=chunk):
                    _one_pass(xs[chunk], outs[chunk], src_r, dst_r, ew_r,
                              acc, zbuf, rows, srcv, dstv, ewv, sid)

    f = pl.kernel(body, out_type=out_type, mesh=mesh, scratch_types=scratch)
    return list(f(*xchunks, src, dst, ew))


BN = 2048


def _layer_mm(aggr_chunks, x_chunks, Wrel, Wroot, brel):
    nci = len(aggr_chunks)
    nco = 4
    din = nci * CW

    def body(*refs):
        aggrs = refs[:nci]
        xs = refs[nci:2 * nci]
        wrel, wroot, b = refs[2 * nci:2 * nci + 3]
        outs = refs[2 * nci + 3:]
        acc = jnp.broadcast_to(b[...], (BN, 512)).astype(jnp.float32)
        for c in range(nci):
            acc = acc + jnp.dot(aggrs[c][...], wrel[pl.ds(c * CW, CW), :],
                                preferred_element_type=jnp.float32)
            acc = acc + jnp.dot(xs[c][...], wroot[pl.ds(c * CW, CW), :],
                                preferred_element_type=jnp.float32)
        acc = jnp.where(acc >= 0, acc, acc * jnp.float32(0.01))
        for c in range(nco):
            outs[c][...] = acc[:, c * CW:(c + 1) * CW]

    grid = (NP // BN,)
    in_specs = (
        [pl.BlockSpec((BN, CW), lambda i: (i, 0)) for _ in range(2 * nci)]
        + [pl.BlockSpec((din, 512), lambda i: (0, 0)),
           pl.BlockSpec((din, 512), lambda i: (0, 0)),
           pl.BlockSpec((1, 512), lambda i: (0, 0))]
    )
    out_specs = [pl.BlockSpec((BN, CW), lambda i: (i, 0))] * nco
    out_shape = [jax.ShapeDtypeStruct((NP, CW), jnp.float32)] * nco
    f = pl.pallas_call(body, grid=grid, in_specs=in_specs,
                       out_specs=out_specs, out_shape=out_shape)
    return list(f(*aggr_chunks, *x_chunks, Wrel, Wroot,
                  brel.reshape(1, 512)))


def _head(x_chunks, W1, b1, W2, b2):
    def body(x0, x1, x2, x3, w1, b1r, w2, b2r, out):
        acc = jnp.broadcast_to(b1r[...], (BN, 512)).astype(jnp.float32)
        for c, xr in enumerate((x0, x1, x2, x3)):
            acc = acc + jnp.dot(xr[...], w1[pl.ds(c * CW, CW), :],
                                preferred_element_type=jnp.float32)
        h = jnp.maximum(acc, 0.0)
        out[...] = (jnp.dot(h, w2[...], preferred_element_type=jnp.float32)
                    + b2r[...])

    grid = (NP // BN,)
    in_specs = (
        [pl.BlockSpec((BN, CW), lambda i: (i, 0)) for _ in range(4)]
        + [pl.BlockSpec((512, 512), lambda i: (0, 0)),
           pl.BlockSpec((1, 512), lambda i: (0, 0)),
           pl.BlockSpec((512, 128), lambda i: (0, 0)),
           pl.BlockSpec((1, 128), lambda i: (0, 0))]
    )
    out_specs = pl.BlockSpec((BN, 128), lambda i: (i, 0))
    out_shape = jax.ShapeDtypeStruct((NP, 128), jnp.float32)
    f = pl.pallas_call(body, grid=grid, in_specs=in_specs,
                       out_specs=out_specs, out_shape=out_shape)
    return f(*x_chunks, W1, b1.reshape(1, 512), W2, b2.reshape(1, 128))


def kernel(x_Entity, x_Transaction, edge_index_e2t, edge_index_t2e,
           edge_weight_e2t, edge_weight_t2e,
           l1_e2t_Wrel, l1_e2t_brel, l1_e2t_Wroot,
           l1_t2e_Wrel, l1_t2e_brel, l1_t2e_Wroot,
           l2_e2t_Wrel, l2_e2t_brel, l2_e2t_Wroot,
           l2_t2e_Wrel, l2_t2e_brel, l2_t2e_Wroot,
           l3_e2t_Wrel, l3_e2t_brel, l3_e2t_Wroot,
           l3_t2e_Wrel, l3_t2e_brel, l3_t2e_Wroot,
           lin1_W, lin1_b, lin2_W, lin2_b):
    src_e2t = edge_index_e2t[0].astype(jnp.int32)
    dst_e2t = edge_index_e2t[1].astype(jnp.int32)
    src_t2e = edge_index_t2e[0].astype(jnp.int32)
    dst_t2e = edge_index_t2e[1].astype(jnp.int32)
    ew_e2t = edge_weight_e2t.astype(jnp.float32)
    ew_t2e = edge_weight_t2e.astype(jnp.float32)

    pad = ((0, NP - N), (0, 0))
    xep = jnp.pad(x_Entity, pad)
    xtp = jnp.pad(x_Transaction, pad)
    xe = [xep[:, :CW], xep[:, CW:]]
    xt = [xtp[:, :CW], xtp[:, CW:]]

    wts = {
        1: (l1_e2t_Wrel, l1_e2t_brel, l1_e2t_Wroot,
            l1_t2e_Wrel, l1_t2e_brel, l1_t2e_Wroot),
        2: (l2_e2t_Wrel, l2_e2t_brel, l2_e2t_Wroot,
            l2_t2e_Wrel, l2_t2e_brel, l2_t2e_Wroot),
        3: (l3_e2t_Wrel, l3_e2t_brel, l3_e2t_Wroot,
            l3_t2e_Wrel, l3_t2e_brel, l3_t2e_Wroot),
    }

    for l in (1, 2, 3):
        (wrel_et, brel_et, wroot_et,
         wrel_te, brel_te, wroot_te) = wts[l]
        if l < 3:
            aggr_t = _segment_sum_sc(xe, src_e2t, dst_e2t, ew_e2t)
        aggr_e = _segment_sum_sc(xt, src_t2e, dst_t2e, ew_t2e)
        new_xe = _layer_mm(aggr_e, xe, wrel_te, wroot_te, brel_te)
        if l < 3:
            xt_new = _layer_mm(aggr_t, xt, wrel_et, wroot_et, brel_et)
            xt = xt_new
        xe = new_xe

    return _head(xe, lin1_W, lin1_b, lin2_W, lin2_b)[:N]


# speedup vs baseline: 5.1115x; 2.6117x over previous
import functools

import jax
import jax.numpy as jnp
from jax import lax
from jax.experimental import pallas as pl
from jax.experimental.pallas import tpu as pltpu
from jax.experimental.pallas import tpu_sc as plsc

N = 10000
NP = 10240
E = 160000
LANES = 16
NSUB = 16
NCORE = 2
EPT = E // NSUB
EB = 80
NBATCH = EPT // EB
RPT = NP // NSUB
CW = 128


_GATHER_DNUMS = lax.GatherDimensionNumbers(
    offset_dims=(), collapsed_slice_dims=(0,), start_index_map=(0,))


def _splat(vec, j):
    idx = jnp.full((LANES, 1), j, jnp.int32)
    return lax.gather(vec, idx, _GATHER_DNUMS, (1,),
                      mode=lax.GatherScatterMode.PROMISE_IN_BOUNDS)


def _scale_rows(rows, ewbuf, b):
    def group(g, carry):
        ewreg = ewbuf[pl.ds(b * EB + g * LANES, LANES)]
        for j in range(LANES):
            sp = _splat(ewreg, j)
            i = g * LANES + j
            for k in range(CW // LANES):
                sl = pl.ds(k * LANES, LANES)
                rows[i, sl] = rows[i, sl] * sp
        return carry
    lax.fori_loop(0, EB // LANES, group, 0)


def _one_pass(xc, outc, dst_r, acc, srcbuf, ewbuf, rows, dstv, sems,
              sid):
    row0 = sid * RPT
    def zfill(i, carry):
        for k in range(CW // LANES):
            rows[0][i, pl.ds(k * LANES, LANES)] = jnp.zeros(
                (LANES,), jnp.float32)
        return carry
    lax.fori_loop(0, EB, zfill, 0)
    gs = sems[0]
    for z in range(RPT // EB):
        pltpu.async_copy(rows[0], acc.at[pl.ds(row0 + z * EB, EB)],
                         gs[z % 2])
    for z in range(RPT // EB):
        pltpu.make_async_copy(
            rows[0], acc.at[pl.ds(row0 + (z % 2) * EB, EB)],
            gs[z % 2]).wait()
    plsc.subcore_barrier()
    ebase = sid * EPT
    gs, ds_, ss = sems

    def gather_desc(b, p):
        return pltpu.make_async_copy(
            xc.at[srcbuf.at[pl.ds(b * EB, EB)]], rows[p], gs[p])

    def dst_desc(b, p):
        return pltpu.make_async_copy(
            dst_r.at[pl.ds(ebase + b * EB, EB)], dstv[p], ds_[p])

    def scat_start(p):
        pltpu.async_copy(rows[p], acc.at[dstv[p]], ss[p], add=True)

    def scat_wait(p):
        pltpu.make_async_copy(rows[p], acc.at[dstv[p]], ss[p]).wait()

    dst_desc(0, 0).start()
    gather_desc(0, 0).start()
    dst_desc(1, 1).start()
    gather_desc(1, 1).start()

    def step(t, carry):
        a = 2 * t
        gather_desc(a, 0).wait()
        _scale_rows(rows[0], ewbuf, a)
        dst_desc(a, 0).wait()
        scat_start(0)
        gather_desc(a + 1, 1).wait()
        _scale_rows(rows[1], ewbuf, a + 1)
        scat_wait(0)
        dst_desc(a + 2, 0).start()
        gather_desc(a + 2, 0).start()
        dst_desc(a + 1, 1).wait()
        scat_start(1)

        @pl.when(t < (NBATCH - 1) // 2 - 1)
        def _():
            scat_wait(1)
            dst_desc(a + 3, 1).start()
            gather_desc(a + 3, 1).start()
        return carry

    lax.fori_loop(0, (NBATCH - 1) // 2, step, 0)
    b_last = NBATCH - 1
    gather_desc(b_last, 0).wait()
    _scale_rows(rows[0], ewbuf, b_last)
    dst_desc(b_last, 0).wait()
    scat_start(0)
    scat_wait(0)
    scat_wait(1)
    plsc.subcore_barrier()
    pltpu.sync_copy(acc.at[pl.ds(row0, RPT)], outc.at[pl.ds(row0, RPT)])
    plsc.subcore_barrier()


def _segment_sum_sc(xchunks, src, dst, ew):
    nc = len(xchunks)
    npc = nc // NCORE
    mesh = plsc.VectorSubcoreMesh(core_axis_name="c", subcore_axis_name="s")
    out_type = tuple(
        jax.ShapeDtypeStruct((NP, CW), jnp.float32) for _ in range(nc))
    scratch = [
        pltpu.VMEM_SHARED((NP, CW), jnp.float32),
        pltpu.VMEM((EPT,), jnp.int32),
        pltpu.VMEM((EPT,), jnp.float32),
        pltpu.VMEM((EB, CW), jnp.float32),
        pltpu.VMEM((EB, CW), jnp.float32),
        pltpu.VMEM((EB,), jnp.int32),
        pltpu.VMEM((EB,), jnp.int32),
        pltpu.SemaphoreType.DMA,
        pltpu.SemaphoreType.DMA,
        pltpu.SemaphoreType.DMA,
        pltpu.SemaphoreType.DMA,
        pltpu.SemaphoreType.DMA,
        pltpu.SemaphoreType.DMA,
    ]

    def body(*refs):
        xs = refs[:nc]
        src_r, dst_r, ew_r = refs[nc:nc + 3]
        outs = refs[nc + 3:2 * nc + 3]
        (acc, srcbuf, ewbuf, rows0, rows1, dstv0, dstv1,
         gs0, gs1, ds0, ds1, ss0, ss1) = refs[2 * nc + 3:]
        rows = (rows0, rows1)
        dstv = (dstv0, dstv1)
        sems = ((gs0, gs1), (ds0, ds1), (ss0, ss1))
        cid = lax.axis_index("c")
        sid = lax.axis_index("s")

        pltpu.sync_copy(src_r.at[pl.ds(sid * EPT, EPT)], srcbuf)
        pltpu.sync_copy(ew_r.at[pl.ds(sid * EPT, EPT)], ewbuf)

        for j in range(npc):
            for core in range(NCORE):
                chunk = core * npc + j

                @pl.when(cid == core)
                def _(chunk=chunk):
                    _one_pass(xs[chunk], outs[chunk], dst_r, acc,
                              srcbuf, ewbuf, rows, dstv, sems, sid)

    f = pl.kernel(body, out_type=out_type, mesh=mesh, scratch_types=scratch)
    return list(f(*xchunks, src, dst, ew))


BN = 2048


def _layer_mm(aggr_chunks, x_chunks, Wrel, Wroot, brel):
    nci = len(aggr_chunks)
    nco = 4
    din = nci * CW

    def body(*refs):
        aggrs = refs[:nci]
        xs = refs[nci:2 * nci]
        wrel, wroot, b = refs[2 * nci:2 * nci + 3]
        outs = refs[2 * nci + 3:]
        acc = jnp.broadcast_to(b[...], (BN, 512)).astype(jnp.float32)
        for c in range(nci):
            acc = acc + jnp.dot(aggrs[c][...], wrel[pl.ds(c * CW, CW), :],
                                preferred_element_type=jnp.float32)
            acc = acc + jnp.dot(xs[c][...], wroot[pl.ds(c * CW, CW), :],
                                preferred_element_type=jnp.float32)
        acc = jnp.where(acc >= 0, acc, acc * jnp.float32(0.01))
        for c in range(nco):
            outs[c][...] = acc[:, c * CW:(c + 1) * CW]

    grid = (NP // BN,)
    in_specs = (
        [pl.BlockSpec((BN, CW), lambda i: (i, 0)) for _ in range(2 * nci)]
        + [pl.BlockSpec((din, 512), lambda i: (0, 0)),
           pl.BlockSpec((din, 512), lambda i: (0, 0)),
           pl.BlockSpec((1, 512), lambda i: (0, 0))]
    )
    out_specs = [pl.BlockSpec((BN, CW), lambda i: (i, 0))] * nco
    out_shape = [jax.ShapeDtypeStruct((NP, CW), jnp.float32)] * nco
    f = pl.pallas_call(body, grid=grid, in_specs=in_specs,
                       out_specs=out_specs, out_shape=out_shape)
    return list(f(*aggr_chunks, *x_chunks, Wrel, Wroot,
                  brel.reshape(1, 512)))


def _head(x_chunks, W1, b1, W2, b2):
    def body(x0, x1, x2, x3, w1, b1r, w2, b2r, out):
        acc = jnp.broadcast_to(b1r[...], (BN, 512)).astype(jnp.float32)
        for c, xr in enumerate((x0, x1, x2, x3)):
            acc = acc + jnp.dot(xr[...], w1[pl.ds(c * CW, CW), :],
                                preferred_element_type=jnp.float32)
        h = jnp.maximum(acc, 0.0)
        out[...] = (jnp.dot(h, w2[...], preferred_element_type=jnp.float32)
                    + b2r[...])

    grid = (NP // BN,)
    in_specs = (
        [pl.BlockSpec((BN, CW), lambda i: (i, 0)) for _ in range(4)]
        + [pl.BlockSpec((512, 512), lambda i: (0, 0)),
           pl.BlockSpec((1, 512), lambda i: (0, 0)),
           pl.BlockSpec((512, 128), lambda i: (0, 0)),
           pl.BlockSpec((1, 128), lambda i: (0, 0))]
    )
    out_specs = pl.BlockSpec((BN, 128), lambda i: (i, 0))
    out_shape = jax.ShapeDtypeStruct((NP, 128), jnp.float32)
    f = pl.pallas_call(body, grid=grid, in_specs=in_specs,
                       out_specs=out_specs, out_shape=out_shape)
    return f(*x_chunks, W1, b1.reshape(1, 512), W2, b2.reshape(1, 128))


def kernel(x_Entity, x_Transaction, edge_index_e2t, edge_index_t2e,
           edge_weight_e2t, edge_weight_t2e,
           l1_e2t_Wrel, l1_e2t_brel, l1_e2t_Wroot,
           l1_t2e_Wrel, l1_t2e_brel, l1_t2e_Wroot,
           l2_e2t_Wrel, l2_e2t_brel, l2_e2t_Wroot,
           l2_t2e_Wrel, l2_t2e_brel, l2_t2e_Wroot,
           l3_e2t_Wrel, l3_e2t_brel, l3_e2t_Wroot,
           l3_t2e_Wrel, l3_t2e_brel, l3_t2e_Wroot,
           lin1_W, lin1_b, lin2_W, lin2_b):
    src_e2t = edge_index_e2t[0].astype(jnp.int32)
    dst_e2t = edge_index_e2t[1].astype(jnp.int32)
    src_t2e = edge_index_t2e[0].astype(jnp.int32)
    dst_t2e = edge_index_t2e[1].astype(jnp.int32)
    ew_e2t = edge_weight_e2t.astype(jnp.float32)
    ew_t2e = edge_weight_t2e.astype(jnp.float32)

    pad = ((0, NP - N), (0, 0))
    xep = jnp.pad(x_Entity, pad)
    xtp = jnp.pad(x_Transaction, pad)
    xe = [xep[:, :CW], xep[:, CW:]]
    xt = [xtp[:, :CW], xtp[:, CW:]]

    wts = {
        1: (l1_e2t_Wrel, l1_e2t_brel, l1_e2t_Wroot,
            l1_t2e_Wrel, l1_t2e_brel, l1_t2e_Wroot),
        2: (l2_e2t_Wrel, l2_e2t_brel, l2_e2t_Wroot,
            l2_t2e_Wrel, l2_t2e_brel, l2_t2e_Wroot),
        3: (l3_e2t_Wrel, l3_e2t_brel, l3_e2t_Wroot,
            l3_t2e_Wrel, l3_t2e_brel, l3_t2e_Wroot),
    }

    for l in (1, 2, 3):
        (wrel_et, brel_et, wroot_et,
         wrel_te, brel_te, wroot_te) = wts[l]
        if l < 3:
            aggr_t = _segment_sum_sc(xe, src_e2t, dst_e2t, ew_e2t)
        aggr_e = _segment_sum_sc(xt, src_t2e, dst_t2e, ew_t2e)
        new_xe = _layer_mm(aggr_e, xe, wrel_te, wroot_te, brel_te)
        if l < 3:
            xt_new = _layer_mm(aggr_t, xt, wrel_et, wroot_et, brel_et)
            xt = xt_new
        xe = new_xe

    return _head(xe, lin1_W, lin1_b, lin2_W, lin2_b)[:N]


# speedup vs baseline: 6.1377x; 1.2008x over previous
import jax
import jax.numpy as jnp
from jax import lax
from jax.experimental import pallas as pl
from jax.experimental.pallas import tpu as pltpu
from jax.experimental.pallas import tpu_sc as plsc

N = 10000
NP = 10240
E = 160000
LANES = 16
NSUB = 16
NCORE = 2
EPT = E // NSUB
EB = 80
NBATCH = EPT // EB
RPT = NP // NSUB
CW = 128
NRING = 4


_GATHER_DNUMS = lax.GatherDimensionNumbers(
    offset_dims=(), collapsed_slice_dims=(0,), start_index_map=(0,))


def _splat(vec, j):
    idx = jnp.full((LANES, 1), j, jnp.int32)
    return lax.gather(vec, idx, _GATHER_DNUMS, (1,),
                      mode=lax.GatherScatterMode.PROMISE_IN_BOUNDS)


def _scale_rows(rows, ewv):
    def group(g, carry):
        ewreg = ewv[pl.ds(g * LANES, LANES)]
        for j in range(LANES):
            sp = _splat(ewreg, j)
            i = g * LANES + j
            for k in range(CW // LANES):
                sl = pl.ds(k * LANES, LANES)
                rows[i, sl] = rows[i, sl] * sp
        return carry
    lax.fori_loop(0, EB // LANES, group, 0)


def _one_pass(xc, outc, src_r, dst_r, ew_r, acc, rows, srcv, dstv, ewv,
              gs, isem, ss, sid):
    row0 = sid * RPT
    ebase = sid * EPT

    def zfill(i, carry):
        for k in range(CW // LANES):
            rows[0][i, pl.ds(k * LANES, LANES)] = jnp.zeros(
                (LANES,), jnp.float32)
        return carry
    lax.fori_loop(0, EB, zfill, 0)
    for z in range(RPT // EB):
        pltpu.async_copy(rows[0], acc.at[pl.ds(row0 + z * EB, EB)],
                         gs[z % 2])
    for z in range(RPT // EB):
        pltpu.make_async_copy(
            rows[0], acc.at[pl.ds(row0 + (z % 2) * EB, EB)],
            gs[z % 2]).wait()
    plsc.subcore_barrier()

    def idx_descs(b, p):
        sl = pl.ds(ebase + b * EB, EB)
        return (pltpu.make_async_copy(src_r.at[sl], srcv[p], isem[p]),
                pltpu.make_async_copy(dst_r.at[sl], dstv[p], isem[p]),
                pltpu.make_async_copy(ew_r.at[sl], ewv[p], isem[p]))

    def idx_start(b, p):
        for d in idx_descs(b, p):
            d.start()

    def idx_wait(b, p):
        for d in idx_descs(b, p):
            d.wait()

    def gather_desc(p):
        return pltpu.make_async_copy(xc.at[srcv[p]], rows[p], gs[p])

    def scat_start(p):
        pltpu.async_copy(rows[p], acc.at[dstv[p]], ss[p], add=True)

    def scat_wait(p):
        pltpu.make_async_copy(rows[p], acc.at[dstv[p]], ss[p]).wait()

    def substep(k, p, w_scat, g_next, i_next2):
        if w_scat:
            scat_wait((p + 2) % NRING)
        if g_next:
            idx_wait(k + 1, (p + 1) % NRING)
            gather_desc((p + 1) % NRING).start()
        if i_next2:
            idx_start(k + 2, (p + 2) % NRING)
        gather_desc(p).wait()
        _scale_rows(rows[p], ewv[p])
        scat_start(p)

    idx_start(0, 0)
    idx_start(1, 1)
    idx_wait(0, 0)
    gather_desc(0).start()
    substep(0, 0, False, True, True)
    substep(1, 1, False, True, True)
    substep(2, 2, True, True, True)
    substep(3, 3, True, True, True)

    def step(t, carry):
        a = NRING * t
        for j in range(NRING):
            substep(a + j, j, True, True, True)
        return carry
    lax.fori_loop(1, (NBATCH - 5) // NRING, step, 0)

    substep(120, 0, True, True, True)
    substep(121, 1, True, True, True)
    substep(122, 2, True, True, True)
    substep(123, 3, True, True, False)
    substep(124, 0, True, False, False)
    scat_wait(3)
    scat_wait(0)

    plsc.subcore_barrier()
    pltpu.sync_copy(acc.at[pl.ds(row0, RPT)], outc.at[pl.ds(row0, RPT)])
    plsc.subcore_barrier()


def _segment_sum_sc(xstack, src, dst, ew):
    nc = xstack.shape[0]
    npc = nc // NCORE
    mesh = plsc.VectorSubcoreMesh(core_axis_name="c", subcore_axis_name="s")
    out_type = jax.ShapeDtypeStruct((nc, NP, CW), jnp.float32)
    scratch = [
        pltpu.VMEM_SHARED((NP, CW), jnp.float32),
    ]
    scratch += [pltpu.VMEM((EB, CW), jnp.float32) for _ in range(NRING)]
    scratch += [pltpu.VMEM((EB,), jnp.int32) for _ in range(NRING)]
    scratch += [pltpu.VMEM((EB,), jnp.int32) for _ in range(NRING)]
    scratch += [pltpu.VMEM((EB,), jnp.float32) for _ in range(NRING)]
    scratch += [pltpu.SemaphoreType.DMA] * (3 * NRING)

    def body(xs_r, src_r, dst_r, ew_r, out_r, *sc):
        acc = sc[0]
        rows = sc[1:1 + NRING]
        srcv = sc[1 + NRING:1 + 2 * NRING]
        dstv = sc[1 + 2 * NRING:1 + 3 * NRING]
        ewv = sc[1 + 3 * NRING:1 + 4 * NRING]
        sems = sc[1 + 4 * NRING:]
        gs = sems[:NRING]
        isem = sems[NRING:2 * NRING]
        ss = sems[2 * NRING:3 * NRING]
        cid = lax.axis_index("c")
        sid = lax.axis_index("s")

        def chunk_pass(j, carry):
            chunk = cid * npc + j
            _one_pass(xs_r.at[chunk], out_r.at[chunk], src_r, dst_r,
                      ew_r, acc, rows, srcv, dstv, ewv, gs, isem, ss,
                      sid)
            return carry
        lax.fori_loop(0, npc, chunk_pass, 0)

    f = pl.kernel(body, out_type=out_type, mesh=mesh, scratch_types=scratch)
    return f(xstack, src, dst, ew)


BN = 2048


def _layer_mm(aggr, xstack, Wrel, Wroot, brel):
    nci = aggr.shape[0]
    nco = 4
    din = nci * CW

    def body(aggr_ref, x_ref, wrel, wroot, b, out_ref):
        acc = jnp.broadcast_to(b[...], (BN, 512)).astype(jnp.float32)
        for c in range(nci):
            acc = acc + jnp.dot(aggr_ref[c], wrel[pl.ds(c * CW, CW), :],
                                preferred_element_type=jnp.float32)
            acc = acc + jnp.dot(x_ref[c], wroot[pl.ds(c * CW, CW), :],
                                preferred_element_type=jnp.float32)
        acc = jnp.where(acc >= 0, acc, acc * jnp.float32(0.01))
        for c in range(nco):
            out_ref[c] = acc[:, c * CW:(c + 1) * CW]

    grid = (NP // BN,)
    in_specs = [
        pl.BlockSpec((nci, BN, CW), lambda i: (0, i, 0)),
        pl.BlockSpec((nci, BN, CW), lambda i: (0, i, 0)),
        pl.BlockSpec((din, 512), lambda i: (0, 0)),
        pl.BlockSpec((din, 512), lambda i: (0, 0)),
        pl.BlockSpec((1, 512), lambda i: (0, 0)),
    ]
    out_specs = pl.BlockSpec((nco, BN, CW), lambda i: (0, i, 0))
    out_shape = jax.ShapeDtypeStruct((nco, NP, CW), jnp.float32)
    f = pl.pallas_call(body, grid=grid, in_specs=in_specs,
                       out_specs=out_specs, out_shape=out_shape)
    return f(aggr, xstack, Wrel, Wroot, brel.reshape(1, 512))


def _head(xstack, W1, b1, W2, b2):
    def body(x_ref, w1, b1r, w2, b2r, out):
        acc = jnp.broadcast_to(b1r[...], (BN, 512)).astype(jnp.float32)
        for c in range(4):
            acc = acc + jnp.dot(x_ref[c], w1[pl.ds(c * CW, CW), :],
                                preferred_element_type=jnp.float32)
        h = jnp.maximum(acc, 0.0)
        out[...] = (jnp.dot(h, w2[...], preferred_element_type=jnp.float32)
                    + b2r[...])

    grid = (NP // BN,)
    in_specs = [
        pl.BlockSpec((4, BN, CW), lambda i: (0, i, 0)),
        pl.BlockSpec((512, 512), lambda i: (0, 0)),
        pl.BlockSpec((1, 512), lambda i: (0, 0)),
        pl.BlockSpec((512, 128), lambda i: (0, 0)),
        pl.BlockSpec((1, 128), lambda i: (0, 0)),
    ]
    out_specs = pl.BlockSpec((BN, 128), lambda i: (i, 0))
    out_shape = jax.ShapeDtypeStruct((NP, 128), jnp.float32)
    f = pl.pallas_call(body, grid=grid, in_specs=in_specs,
                       out_specs=out_specs, out_shape=out_shape)
    return f(xstack, W1, b1.reshape(1, 512), W2, b2.reshape(1, 128))


def kernel(x_Entity, x_Transaction, edge_index_e2t, edge_index_t2e,
           edge_weight_e2t, edge_weight_t2e,
           l1_e2t_Wrel, l1_e2t_brel, l1_e2t_Wroot,
           l1_t2e_Wrel, l1_t2e_brel, l1_t2e_Wroot,
           l2_e2t_Wrel, l2_e2t_brel, l2_e2t_Wroot,
           l2_t2e_Wrel, l2_t2e_brel, l2_t2e_Wroot,
           l3_e2t_Wrel, l3_e2t_brel, l3_e2t_Wroot,
           l3_t2e_Wrel, l3_t2e_brel, l3_t2e_Wroot,
           lin1_W, lin1_b, lin2_W, lin2_b):
    src_e2t = edge_index_e2t[0].astype(jnp.int32)
    dst_e2t = edge_index_e2t[1].astype(jnp.int32)
    src_t2e = edge_index_t2e[0].astype(jnp.int32)
    dst_t2e = edge_index_t2e[1].astype(jnp.int32)
    ew_e2t = edge_weight_e2t.astype(jnp.float32)
    ew_t2e = edge_weight_t2e.astype(jnp.float32)

    def stack2(x):
        xp = jnp.pad(x, ((0, NP - N), (0, 0)))
        return jnp.stack([xp[:, :CW], xp[:, CW:]])

    xe = stack2(x_Entity)
    xt = stack2(x_Transaction)

    wts = {
        1: (l1_e2t_Wrel, l1_e2t_brel, l1_e2t_Wroot,
            l1_t2e_Wrel, l1_t2e_brel, l1_t2e_Wroot),
        2: (l2_e2t_Wrel, l2_e2t_brel, l2_e2t_Wroot,
            l2_t2e_Wrel, l2_t2e_brel, l2_t2e_Wroot),
        3: (l3_e2t_Wrel, l3_e2t_brel, l3_e2t_Wroot,
            l3_t2e_Wrel, l3_t2e_brel, l3_t2e_Wroot),
    }

    for l in (1, 2, 3):
        (wrel_et, brel_et, wroot_et,
         wrel_te, brel_te, wroot_te) = wts[l]
        if l < 3:
            aggr_t = _segment_sum_sc(xe, src_e2t, dst_e2t, ew_e2t)
        aggr_e = _segment_sum_sc(xt, src_t2e, dst_t2e, ew_t2e)
        new_xe = _layer_mm(aggr_e, xe, wrel_te, wroot_te, brel_te)
        if l < 3:
            xt = _layer_mm(aggr_t, xt, wrel_et, wroot_et, brel_et)
        xe = new_xe

    return _head(xe, lin1_W, lin1_b, lin2_W, lin2_b)[:N]
